# baseline (speedup 1.0000x reference)
import functools

import jax
import jax.numpy as jnp
from jax import lax
from jax.experimental import pallas as pl
from jax.experimental.pallas import tpu as pltpu
from jax.experimental.pallas import tpu_sc as plsc

B = 16384
F = 14
V = 100000
D = 32
NUM_NUMERIC = 64
NUM_CLASSES = 2

NC = 2
NS = 16
NW = NC * NS
ROWS_PW = B // NW
C = 128
NCHUNK = ROWS_PW // C
IPC = C * F
NIV = IPC // 128
XROWS_PW = ROWS_PW * F // 128

_mesh = plsc.VectorSubcoreMesh(core_axis_name="c", subcore_axis_name="s")


@functools.partial(
    pl.kernel,
    mesh=_mesh,
    compiler_params=pltpu.CompilerParams(use_tc_tiling_on_sc=False),
    out_type=jax.ShapeDtypeStruct((B, D), jnp.float32),
    scratch_types=[
        pltpu.VMEM((NIV, 128), jnp.int32),
        pltpu.VMEM((XROWS_PW, 128), jnp.int32),
        pltpu.VMEM((NIV, 128), jnp.int32),
        pltpu.VMEM((IPC, D), jnp.float32),
        pltpu.VMEM((C, D), jnp.float32),
        pltpu.SemaphoreType.DMA,
    ],
)
def _sc_embed(xcat_hbm, tab_hbm, off_hbm, out_hbm,
              off_v, xcat_v, idx_v, rows_v, emb_v, sem):
    wid = lax.axis_index("s") * NC + lax.axis_index("c")
    pltpu.sync_copy(off_hbm, off_v)
    for ch in range(NCHUNK):
        xrow0 = wid * XROWS_PW + ch * NIV
        out0 = wid * ROWS_PW + ch * C
        pltpu.sync_copy(xcat_hbm.at[pl.ds(xrow0, NIV)], xcat_v)
        for r in range(NIV):
            for k in range(128 // 16):
                s = pl.ds(k * 16, 16)
                idx_v[r, s] = xcat_v[r, s] + off_v[r, s]
        cps = [
            pltpu.async_copy(tab_hbm.at[idx_v.at[r]],
                             rows_v.at[pl.ds(r * 128, 128)], sem)
            for r in range(NIV)
        ]
        for cp in cps:
            cp.wait()

        def _red(c, carry):
            base = c * F
            for h in range(D // 16):
                s = pl.ds(h * 16, 16)
                acc = rows_v[base, s]
                for f in range(1, F):
                    acc = acc + rows_v[base + f, s]
                emb_v[c, s] = acc
            return carry

        lax.fori_loop(0, C, _red, 0)
        pltpu.sync_copy(emb_v, out_hbm.at[pl.ds(out0, C)])


_BLK = 2048


def _dense_body(emb_ref, xn_ref, wn_ref, bn_ref, wo_ref, bo_ref, out_ref):
    h = jnp.dot(xn_ref[...], wn_ref[...], preferred_element_type=jnp.float32)
    h = h + bn_ref[...] + emb_ref[...]
    out_ref[...] = jnp.dot(h, wo_ref[...],
                           preferred_element_type=jnp.float32) + bo_ref[...]


def _dense(emb, x_num, W_num, b_num, W_out, b_out):
    grid = (B // _BLK,)
    return pl.pallas_call(
        _dense_body,
        grid=grid,
        in_specs=[
            pl.BlockSpec((_BLK, D), lambda i: (i, 0)),
            pl.BlockSpec((_BLK, NUM_NUMERIC), lambda i: (i, 0)),
            pl.BlockSpec((NUM_NUMERIC, D), lambda i: (0, 0)),
            pl.BlockSpec((1, D), lambda i: (0, 0)),
            pl.BlockSpec((D, NUM_CLASSES), lambda i: (0, 0)),
            pl.BlockSpec((1, NUM_CLASSES), lambda i: (0, 0)),
        ],
        out_specs=pl.BlockSpec((_BLK, NUM_CLASSES), lambda i: (i, 0)),
        out_shape=jax.ShapeDtypeStruct((B, NUM_CLASSES), jnp.float32),
    )(emb, x_num, W_num, b_num, W_out, b_out)


def kernel(x_cat, x_num, tables, W_num, b_num, W_out, b_out):
    tab_flat = tables.reshape(F * V, D)
    xcat2d = x_cat.reshape(B * F // 128, 128)
    off2d = ((jnp.arange(IPC, dtype=jnp.int32) % F) * V).reshape(NIV, 128)
    emb = _sc_embed(xcat2d, tab_flat, off2d)
    return _dense(emb, x_num, W_num, b_num.reshape(1, D),
                  W_out, b_out.reshape(1, NUM_CLASSES))


# speedup vs baseline: 1.0261x; 1.0261x over previous
import functools

import jax
import jax.numpy as jnp
from jax import lax
from jax.experimental import pallas as pl
from jax.experimental.pallas import tpu as pltpu
from jax.experimental.pallas import tpu_sc as plsc

B = 16384
F = 14
V = 100000
D = 32
NUM_NUMERIC = 64
NUM_CLASSES = 2

NC = 2
NS = 16
NW = NC * NS
ROWS_PW = B // NW
C = 128
NCHUNK = ROWS_PW // C

_mesh = plsc.VectorSubcoreMesh(core_axis_name="c", subcore_axis_name="s")


@functools.partial(
    pl.kernel,
    mesh=_mesh,
    compiler_params=pltpu.CompilerParams(use_tc_tiling_on_sc=False),
    out_type=jax.ShapeDtypeStruct((B, D), jnp.float32),
    scratch_types=[
        pltpu.VMEM((F, ROWS_PW), jnp.int32),
        pltpu.VMEM((F * C, D), jnp.float32),
        pltpu.VMEM((C, D), jnp.float32),
        pltpu.SemaphoreType.DMA,
    ],
)
def _sc_embed(xcat_hbm, tab_hbm, out_hbm, xcat_v, rows_v, emb_v, sem):
    wid = lax.axis_index("s") * NC + lax.axis_index("c")
    pltpu.sync_copy(xcat_hbm.at[:, pl.ds(wid * ROWS_PW, ROWS_PW)], xcat_v)
    for ch in range(NCHUNK):
        out0 = wid * ROWS_PW + ch * C
        cps = [
            pltpu.async_copy(
                tab_hbm.at[f].at[xcat_v.at[f, pl.ds(ch * C, C)]],
                rows_v.at[pl.ds(f * C, C)], sem)
            for f in range(F)
        ]
        for cp in cps:
            cp.wait()

        def _red(c, carry):
            for h in range(D // 16):
                s = pl.ds(h * 16, 16)
                acc = rows_v[c, s]
                for f in range(1, F):
                    acc = acc + rows_v[f * C + c, s]
                emb_v[c, s] = acc
            return carry

        lax.fori_loop(0, C, _red, 0)
        pltpu.sync_copy(emb_v, out_hbm.at[pl.ds(out0, C)])


_BLK = 2048


def _dense_body(emb_ref, xn_ref, wn_ref, bn_ref, wo_ref, bo_ref, out_ref):
    h = jnp.dot(xn_ref[...], wn_ref[...], preferred_element_type=jnp.float32)
    h = h + bn_ref[...] + emb_ref[...]
    out_ref[...] = jnp.dot(h, wo_ref[...],
                           preferred_element_type=jnp.float32) + bo_ref[...]


def _dense(emb, x_num, W_num, b_num, W_out, b_out):
    grid = (B // _BLK,)
    return pl.pallas_call(
        _dense_body,
        grid=grid,
        in_specs=[
            pl.BlockSpec((_BLK, D), lambda i: (i, 0)),
            pl.BlockSpec((_BLK, NUM_NUMERIC), lambda i: (i, 0)),
            pl.BlockSpec((NUM_NUMERIC, D), lambda i: (0, 0)),
            pl.BlockSpec((1, D), lambda i: (0, 0)),
            pl.BlockSpec((D, NUM_CLASSES), lambda i: (0, 0)),
            pl.BlockSpec((1, NUM_CLASSES), lambda i: (0, 0)),
        ],
        out_specs=pl.BlockSpec((_BLK, NUM_CLASSES), lambda i: (i, 0)),
        out_shape=jax.ShapeDtypeStruct((B, NUM_CLASSES), jnp.float32),
    )(emb, x_num, W_num, b_num, W_out, b_out)


def kernel(x_cat, x_num, tables, W_num, b_num, W_out, b_out):
    xcat_t = x_cat.T
    emb = _sc_embed(xcat_t, tables)
    return _dense(emb, x_num, W_num, b_num.reshape(1, D),
                  W_out, b_out.reshape(1, NUM_CLASSES))


# speedup vs baseline: 1.1259x; 1.0973x over previous
import functools

import jax
import jax.numpy as jnp
from jax import lax
from jax.experimental import pallas as pl
from jax.experimental.pallas import tpu as pltpu
from jax.experimental.pallas import tpu_sc as plsc

B = 16384
F = 14
V = 100000
D = 32
NUM_NUMERIC = 64
NUM_CLASSES = 2

NC = 2
NS = 16
NW = NC * NS
ROWS_PW = B // NW
C = 128
NCHUNK = ROWS_PW // C

_mesh = plsc.VectorSubcoreMesh(core_axis_name="c", subcore_axis_name="s")


@functools.partial(
    pl.kernel,
    mesh=_mesh,
    compiler_params=pltpu.CompilerParams(use_tc_tiling_on_sc=False),
    out_type=jax.ShapeDtypeStruct((D, B), jnp.float32),
    scratch_types=[
        pltpu.VMEM((F, ROWS_PW), jnp.int32),
        pltpu.VMEM((D, C), jnp.float32),
        pltpu.VMEM((D, C), jnp.float32),
        pltpu.SemaphoreType.DMA,
    ],
)
def _sc_embed(xcat_hbm, tab_hbm, out_hbm, xcat_v, gat_v, emb_v, sem):
    wid = lax.axis_index("s") * NC + lax.axis_index("c")
    b0 = wid * ROWS_PW
    pltpu.sync_copy(xcat_hbm.at[:, pl.ds(b0, ROWS_PW)], xcat_v)
    for ch in range(NCHUNK):
        def _field(f, carry):
            idx = xcat_v.at[f, pl.ds(ch * C, C)]
            cps = [
                pltpu.async_copy(tab_hbm.at[f, d].at[idx], gat_v.at[d], sem)
                for d in range(D)
            ]
            for cp in cps:
                cp.wait()
            for d in range(D):
                for k in range(C // 16):
                    s = pl.ds(k * 16, 16)
                    @pl.when(f == 0)
                    def _():
                        emb_v[d, s] = gat_v[d, s]
                    @pl.when(f != 0)
                    def _():
                        emb_v[d, s] = emb_v[d, s] + gat_v[d, s]
            return carry

        lax.fori_loop(0, F, _field, 0)
        pltpu.sync_copy(emb_v, out_hbm.at[:, pl.ds(b0 + ch * C, C)])


_BLK = 2048


def _dense_body(emb_ref, xn_ref, wn_ref, bn_ref, wo_ref, bo_ref, out_ref):
    h = jnp.dot(wn_ref[...], xn_ref[...], preferred_element_type=jnp.float32)
    h = h + bn_ref[...] + emb_ref[...]
    out_ref[...] = jnp.dot(wo_ref[...], h,
                           preferred_element_type=jnp.float32) + bo_ref[...]


def _dense(emb_t, xn_t, WnT, b_num, WoT, b_out):
    grid = (B // _BLK,)
    return pl.pallas_call(
        _dense_body,
        grid=grid,
        in_specs=[
            pl.BlockSpec((D, _BLK), lambda i: (0, i)),
            pl.BlockSpec((NUM_NUMERIC, _BLK), lambda i: (0, i)),
            pl.BlockSpec((D, NUM_NUMERIC), lambda i: (0, 0)),
            pl.BlockSpec((D, 1), lambda i: (0, 0)),
            pl.BlockSpec((NUM_CLASSES, D), lambda i: (0, 0)),
            pl.BlockSpec((NUM_CLASSES, 1), lambda i: (0, 0)),
        ],
        out_specs=pl.BlockSpec((NUM_CLASSES, _BLK), lambda i: (0, i)),
        out_shape=jax.ShapeDtypeStruct((NUM_CLASSES, B), jnp.float32),
    )(emb_t, xn_t, WnT, b_num, WoT, b_out)


def kernel(x_cat, x_num, tables, W_num, b_num, W_out, b_out):
    xcat_t = x_cat.T
    tab_t = tables.transpose(0, 2, 1)
    emb_t = _sc_embed(xcat_t, tab_t)
    out_t = _dense(emb_t, x_num.T, W_num.T, b_num.reshape(D, 1),
                   W_out.T, b_out.reshape(NUM_CLASSES, 1))
    return out_t.T


# speedup vs baseline: 1.2932x; 1.1486x over previous
import functools

import jax
import jax.numpy as jnp
from jax import lax
from jax.experimental import pallas as pl
from jax.experimental.pallas import tpu as pltpu
from jax.experimental.pallas import tpu_sc as plsc

B = 16384
F = 14
V = 100000
D = 32
NUM_NUMERIC = 64
NUM_CLASSES = 2

NC = 2
NS = 16
NW = NC * NS
ROWS_PW = B // NW
C = 128
NCHUNK = ROWS_PW // C

_mesh = plsc.VectorSubcoreMesh(core_axis_name="c", subcore_axis_name="s")


@functools.partial(
    pl.kernel,
    mesh=_mesh,
    compiler_params=pltpu.CompilerParams(use_tc_tiling_on_sc=False),
    out_type=jax.ShapeDtypeStruct((D, B), jnp.float32),
    scratch_types=[
        pltpu.VMEM((F, ROWS_PW), jnp.int32),
        pltpu.VMEM((2, D, C), jnp.float32),
        pltpu.VMEM((D, C), jnp.float32),
        pltpu.SemaphoreType.DMA((2,)),
    ],
)
def _sc_embed(xcat_hbm, tab_hbm, out_hbm, xcat_v, gat_v, emb_v, sem):
    wid = lax.axis_index("s") * NC + lax.axis_index("c")
    b0 = wid * ROWS_PW
    pltpu.sync_copy(xcat_hbm.at[:, pl.ds(b0, ROWS_PW)], xcat_v)

    def _fire(f, ch, par):
        idx = xcat_v.at[f, pl.ds(ch * C, C)]
        for d in range(D):
            pltpu.async_copy(tab_hbm.at[f, d].at[idx], gat_v.at[par, d],
                             sem.at[par])

    def _chunk(ch, carry):
        for d in range(D):
            for k in range(C // 16):
                emb_v[d, pl.ds(k * 16, 16)] = jnp.zeros((16,), jnp.float32)
        _fire(0, ch, 0)

        def _field(f, carry2):
            par = lax.rem(f, 2)

            @pl.when(f < F - 1)
            def _():
                _fire(f + 1, ch, 1 - par)

            pltpu.make_async_copy(tab_hbm.at[0, :, pl.ds(0, C)],
                                  gat_v.at[par], sem.at[par]).wait()
            for d in range(D):
                for k in range(C // 16):
                    s = pl.ds(k * 16, 16)
                    plsc.addupdate(emb_v.at[d, s], gat_v[par, d, s])
            return carry2

        lax.fori_loop(0, F, _field, 0)
        pltpu.sync_copy(emb_v, out_hbm.at[:, pl.ds(b0 + ch * C, C)])
        return carry

    lax.fori_loop(0, NCHUNK, _chunk, 0)


_BLK = 2048


def _dense_body(emb_ref, xn_ref, wn_ref, bn_ref, wo_ref, bo_ref, out_ref):
    h = jnp.dot(wn_ref[...], xn_ref[...], preferred_element_type=jnp.float32)
    h = h + bn_ref[...] + emb_ref[...]
    out_ref[...] = jnp.dot(wo_ref[...], h,
                           preferred_element_type=jnp.float32) + bo_ref[...]


def _dense(emb_t, xn_t, WnT, b_num, WoT, b_out):
    grid = (B // _BLK,)
    return pl.pallas_call(
        _dense_body,
        grid=grid,
        in_specs=[
            pl.BlockSpec((D, _BLK), lambda i: (0, i)),
            pl.BlockSpec((NUM_NUMERIC, _BLK), lambda i: (0, i)),
            pl.BlockSpec((D, NUM_NUMERIC), lambda i: (0, 0)),
            pl.BlockSpec((D, 1), lambda i: (0, 0)),
            pl.BlockSpec((NUM_CLASSES, D), lambda i: (0, 0)),
            pl.BlockSpec((NUM_CLASSES, 1), lambda i: (0, 0)),
        ],
        out_specs=pl.BlockSpec((NUM_CLASSES, _BLK), lambda i: (0, i)),
        out_shape=jax.ShapeDtypeStruct((NUM_CLASSES, B), jnp.float32),
    )(emb_t, xn_t, WnT, b_num, WoT, b_out)


def kernel(x_cat, x_num, tables, W_num, b_num, W_out, b_out):
    xcat_t = x_cat.T
    tab_t = tables.transpose(0, 2, 1)
    emb_t = _sc_embed(xcat_t, tab_t)
    out_t = _dense(emb_t, x_num.T, W_num.T, b_num.reshape(D, 1),
                   W_out.T, b_out.reshape(NUM_CLASSES, 1))
    return out_t.T


# speedup vs baseline: 1.7122x; 1.3240x over previous
import functools

import jax
import jax.numpy as jnp
from jax import lax
from jax.experimental import pallas as pl
from jax.experimental.pallas import tpu as pltpu
from jax.experimental.pallas import tpu_sc as plsc

B = 16384
F = 14
V = 100000
D = 32
NUM_NUMERIC = 64
NUM_CLASSES = 2

NC = 2
NS = 16
NW = NC * NS
ROWS_PW = B // NW
C = 128
NCHUNK = ROWS_PW // C

VP = 100096
PLANE = D * VP
TABLEN = F * PLANE

_mesh = plsc.VectorSubcoreMesh(core_axis_name="c", subcore_axis_name="s")


def _strip_body(in_ref, out_ref):
    for r in range(8):
        out_ref[pl.ds(r * VP, V)] = in_ref[0, r, :]


def _strip(tab_t):
    return pl.pallas_call(
        _strip_body,
        grid=(F, D // 8),
        in_specs=[pl.BlockSpec((1, 8, V), lambda f, j: (f, j, 0))],
        out_specs=pl.BlockSpec((8 * VP,), lambda f, j: (f * (D // 8) + j,)),
        out_shape=jax.ShapeDtypeStruct((TABLEN,), jnp.float32),
    )(tab_t)


@functools.partial(
    pl.kernel,
    mesh=_mesh,
    compiler_params=pltpu.CompilerParams(use_tc_tiling_on_sc=False),
    out_type=jax.ShapeDtypeStruct((D, B), jnp.float32),
    scratch_types=[
        pltpu.VMEM((F, ROWS_PW), jnp.int32),
        pltpu.VMEM((2, D * C), jnp.float32),
        pltpu.VMEM((D, C), jnp.float32),
        pltpu.SemaphoreType.DMA((2,)),
    ],
)
def _sc_embed(xcat_hbm, tab_hbm, out_hbm, xcat_v, gat_v, emb_v, sem):
    wid = lax.axis_index("s") * NC + lax.axis_index("c")
    b0 = wid * ROWS_PW
    pltpu.sync_copy(xcat_hbm.at[:, pl.ds(b0, ROWS_PW)], xcat_v)

    def _fire(f, ch, par):
        idx = xcat_v.at[f, pl.ds(ch * C, C)]
        base = f * PLANE
        for d in range(D):
            pltpu.async_copy(
                tab_hbm.at[pl.ds(base + d * VP, V)].at[idx],
                gat_v.at[par, pl.ds(d * C, C)], sem.at[par])

    def _chunk(ch, carry):
        for d in range(D):
            for k in range(C // 16):
                emb_v[d, pl.ds(k * 16, 16)] = jnp.zeros((16,), jnp.float32)
        _fire(0, ch, 0)

        def _field(f, carry2):
            par = lax.rem(f, 2)

            @pl.when(f < F - 1)
            def _():
                _fire(f + 1, ch, 1 - par)

            pltpu.make_async_copy(tab_hbm.at[pl.ds(0, D * C)],
                                  gat_v.at[par], sem.at[par]).wait()
            for d in range(D):
                for k in range(C // 16):
                    s = pl.ds(d * C + k * 16, 16)
                    plsc.addupdate(emb_v.at[d, pl.ds(k * 16, 16)],
                                   gat_v[par, s])
            return carry2

        lax.fori_loop(0, F, _field, 0)
        pltpu.sync_copy(emb_v, out_hbm.at[:, pl.ds(b0 + ch * C, C)])
        return carry

    lax.fori_loop(0, NCHUNK, _chunk, 0)


_BLK = 2048


def _dense_body(emb_ref, xn_ref, wn_ref, bn_ref, wo_ref, bo_ref, out_ref):
    h = jnp.dot(wn_ref[...], xn_ref[...], preferred_element_type=jnp.float32)
    h = h + bn_ref[...] + emb_ref[...]
    out_ref[...] = jnp.dot(wo_ref[...], h,
                           preferred_element_type=jnp.float32) + bo_ref[...]


def _dense(emb_t, xn_t, WnT, b_num, WoT, b_out):
    grid = (B // _BLK,)
    return pl.pallas_call(
        _dense_body,
        grid=grid,
        in_specs=[
            pl.BlockSpec((D, _BLK), lambda i: (0, i)),
            pl.BlockSpec((NUM_NUMERIC, _BLK), lambda i: (0, i)),
            pl.BlockSpec((D, NUM_NUMERIC), lambda i: (0, 0)),
            pl.BlockSpec((D, 1), lambda i: (0, 0)),
            pl.BlockSpec((NUM_CLASSES, D), lambda i: (0, 0)),
            pl.BlockSpec((NUM_CLASSES, 1), lambda i: (0, 0)),
        ],
        out_specs=pl.BlockSpec((NUM_CLASSES, _BLK), lambda i: (0, i)),
        out_shape=jax.ShapeDtypeStruct((NUM_CLASSES, B), jnp.float32),
    )(emb_t, xn_t, WnT, b_num, WoT, b_out)


def kernel(x_cat, x_num, tables, W_num, b_num, W_out, b_out):
    xcat_t = x_cat.T
    tab_t = tables.transpose(0, 2, 1)
    tab_flat = _strip(tab_t)
    emb_t = _sc_embed(xcat_t, tab_flat)
    out_t = _dense(emb_t, x_num.T, W_num.T, b_num.reshape(D, 1),
                   W_out.T, b_out.reshape(NUM_CLASSES, 1))
    return out_t.T
